# baseline (speedup 1.0000x reference)
import functools

import jax
import jax.numpy as jnp
from jax import lax
from jax.experimental import pallas as pl
from jax.experimental.pallas import tpu as pltpu
from jax.experimental.pallas import tpu_sc as plsc


def _scores_body(a_ref, wb_ref, o_ref):
    eps = 1e-10

    def r(k):
        return a_ref[k]

    def w(k):
        return wb_ref[k]

    z = w(0) * r(0) + w(1) * r(1) + w(2) * r(2) + w(3) * r(3)
    z = z + w(4) * jnp.log(r(4))
    for j in range(9):
        x = r(6 + j)
        z = z + w(5 + j) * (jnp.log(jnp.abs(x) + eps) * jnp.sign(x))
    z = z + w(14) * (jnp.sqrt(r(7)) / (jnp.sqrt(r(6)) + eps))
    z = z + w(15) * jnp.cos(r(5))
    z = z + w(16) * jnp.sin(r(5))
    z = z + w(17)
    o_ref[...] = 1.0 / (1.0 + jnp.exp(-z))


def _compute_scores(attr_t, weight, bias, *, interpret=False):
    _, rows, lanes = attr_t.shape
    sub = 32
    wb = jnp.concatenate([weight[:, 0], bias])
    out = pl.pallas_call(
        _scores_body,
        grid=(rows // sub,),
        in_specs=[
            pl.BlockSpec((15, sub, lanes), lambda i: (0, i, 0)),
            pl.BlockSpec(memory_space=pltpu.SMEM),
        ],
        out_specs=pl.BlockSpec((sub, lanes), lambda i: (i, 0)),
        out_shape=jax.ShapeDtypeStruct((rows, lanes), jnp.float32),
        interpret=interpret,
    )(attr_t, wb)
    return out


def _gather_mul_sc(scores, x_flat, idx_flat, *, chunk=8192, interpret=False):
    total = x_flat.shape[0]
    n_cc = scores.shape[0]
    try:
        info = plsc.get_sparse_core_info()
        nc, ns = info.num_cores, info.num_subcores
    except ValueError:
        nc, ns = 2, 16
    nw = nc * ns
    per_w = total // nw
    assert total % nw == 0 and per_w % chunk == 0
    n_chunks = per_w // chunk
    mesh = plsc.VectorSubcoreMesh(
        core_axis_name="c", subcore_axis_name="s", num_cores=nc, num_subcores=ns)

    assert n_chunks % 2 == 0 and n_cc % ns == 0

    cs = (chunk * 5) // 8
    ch = chunk - cs
    assert cs % 16 == 0 and ch % 16 == 0

    def body(scores_hbm, x_hbm, idx_hbm, out_hbm, tab_sh, idxs0, idxs1,
             idxh0, idxh1, x0, x1, gs0, gs1, gh0, gh1,
             sem_i, sem_x, sem_gs, sem_gh, sem_o):
        c = lax.axis_index("c")
        s = lax.axis_index("s")
        wid = s * nc + c
        base = wid * per_w
        idxs_b = (idxs0, idxs1)
        idxh_b = (idxh0, idxh1)
        x_b = (x0, x1)
        gs_b = (gs0, gs1)
        gh_b = (gh0, gh1)

        def start_in(k, b):
            off = base + k * chunk
            pltpu.async_copy(idx_hbm.at[pl.ds(off, cs)], idxs_b[b],
                             sem_i.at[b])
            pltpu.async_copy(idx_hbm.at[pl.ds(off + cs, ch)], idxh_b[b],
                             sem_i.at[b])
            pltpu.async_copy(x_hbm.at[pl.ds(off, chunk)], x_b[b], sem_x.at[b])

        def wait_in(b):
            pltpu.make_async_copy(idx_hbm.at[pl.ds(0, cs)], idxs_b[b],
                                  sem_i.at[b]).wait()
            pltpu.make_async_copy(idx_hbm.at[pl.ds(0, ch)], idxh_b[b],
                                  sem_i.at[b]).wait()
            pltpu.make_async_copy(x_hbm.at[pl.ds(0, chunk)], x_b[b],
                                  sem_x.at[b]).wait()

        def wait_out(b):
            pltpu.make_async_copy(gs_b[b], out_hbm.at[pl.ds(0, cs)],
                                  sem_o.at[b]).wait()
            pltpu.make_async_copy(gh_b[b], out_hbm.at[pl.ds(0, ch)],
                                  sem_o.at[b]).wait()

        start_in(0, 0)
        start_in(1, 1)
        seg = n_cc // ns
        pltpu.sync_copy(scores_hbm.at[pl.ds(s * seg, seg)],
                        tab_sh.at[pl.ds(s * seg, seg)])
        plsc.subcore_barrier()

        def chunk_pair(g, carry):
            for b in range(2):
                k = 2 * g + b
                wait_in(b)

                @pl.when(k >= 2)
                def _():
                    wait_out(b)

                pltpu.async_copy(tab_sh.at[idxs_b[b]], gs_b[b], sem_gs.at[b])
                pltpu.async_copy(scores_hbm.at[idxh_b[b]], gh_b[b],
                                 sem_gh.at[b])
                pltpu.make_async_copy(tab_sh.at[idxs_b[b]], gs_b[b],
                                      sem_gs.at[b]).wait()
                pltpu.make_async_copy(scores_hbm.at[idxh_b[b]], gh_b[b],
                                      sem_gh.at[b]).wait()

                def mul_s(i, carry2):
                    sl = pl.ds(i * 16, 16)
                    gs_b[b][sl] = gs_b[b][sl] * x_b[b][sl]
                    return carry2

                def mul_h(i, carry2):
                    sl = pl.ds(i * 16, 16)
                    gh_b[b][sl] = gh_b[b][sl] * x_b[b][pl.ds(cs + i * 16, 16)]
                    return carry2

                lax.fori_loop(0, cs // 16, mul_s, 0, unroll=8)
                lax.fori_loop(0, ch // 16, mul_h, 0, unroll=8)
                off = base + k * chunk
                pltpu.async_copy(gs_b[b], out_hbm.at[pl.ds(off, cs)],
                                 sem_o.at[b])
                pltpu.async_copy(gh_b[b], out_hbm.at[pl.ds(off + cs, ch)],
                                 sem_o.at[b])

                @pl.when(k + 2 < n_chunks)
                def _():
                    start_in(k + 2, b)

            return carry

        lax.fori_loop(0, n_chunks // 2, chunk_pair, 0)
        wait_out(0)
        wait_out(1)

    fn = pl.kernel(
        body,
        out_type=jax.ShapeDtypeStruct((total,), jnp.float32),
        mesh=mesh,
        scratch_types=[
            pltpu.VMEM_SHARED((n_cc,), jnp.float32),
            pltpu.VMEM((cs,), jnp.int32),
            pltpu.VMEM((cs,), jnp.int32),
            pltpu.VMEM((ch,), jnp.int32),
            pltpu.VMEM((ch,), jnp.int32),
            pltpu.VMEM((chunk,), jnp.float32),
            pltpu.VMEM((chunk,), jnp.float32),
            pltpu.VMEM((cs,), jnp.float32),
            pltpu.VMEM((cs,), jnp.float32),
            pltpu.VMEM((ch,), jnp.float32),
            pltpu.VMEM((ch,), jnp.float32),
            pltpu.SemaphoreType.DMA((2,)),
            pltpu.SemaphoreType.DMA((2,)),
            pltpu.SemaphoreType.DMA((2,)),
            pltpu.SemaphoreType.DMA((2,)),
            pltpu.SemaphoreType.DMA((2,)),
        ],
        interpret=interpret,
    )
    return fn(scores, x_flat, idx_flat)


def kernel(input, attributes, pixel_cc, weight, bias):
    h, w = input.shape
    n = attributes.shape[0]
    n_pad = 1 << 20
    rows = lanes = 1024
    attr_p = jnp.concatenate(
        [attributes, jnp.ones((n_pad - n, 15), jnp.float32)], axis=0)
    attr_t = attr_p.T.reshape(15, rows, lanes)
    scores = _compute_scores(attr_t, weight, bias).reshape(-1)
    idx = pixel_cc.reshape(-1).astype(jnp.int32)
    x = input.reshape(-1)
    out = _gather_mul_sc(scores, x, idx)
    return out.reshape(h, w)


# speedup vs baseline: 1.1696x; 1.1696x over previous
import functools

import jax
import jax.numpy as jnp
from jax import lax
from jax.experimental import pallas as pl
from jax.experimental.pallas import tpu as pltpu
from jax.experimental.pallas import tpu_sc as plsc


def _scores_body(a_ref, wb_ref, o_ref):
    eps = 1e-10

    def r(k):
        return a_ref[k]

    def w(k):
        return wb_ref[k]

    z = w(0) * r(0) + w(1) * r(1) + w(2) * r(2) + w(3) * r(3)
    z = z + w(4) * jnp.log(r(4))
    for j in range(9):
        x = r(6 + j)
        z = z + w(5 + j) * (jnp.log(jnp.abs(x) + eps) * jnp.sign(x))
    z = z + w(14) * (jnp.sqrt(r(7)) / (jnp.sqrt(r(6)) + eps))
    z = z + w(15) * jnp.cos(r(5))
    z = z + w(16) * jnp.sin(r(5))
    z = z + w(17)
    o_ref[...] = 1.0 / (1.0 + jnp.exp(-z))


def _compute_scores(attr_t, weight, bias, *, interpret=False):
    _, rows, lanes = attr_t.shape
    sub = 32
    wb = jnp.concatenate([weight[:, 0], bias])
    out = pl.pallas_call(
        _scores_body,
        grid=(rows // sub,),
        in_specs=[
            pl.BlockSpec((15, sub, lanes), lambda i: (0, i, 0)),
            pl.BlockSpec(memory_space=pltpu.SMEM),
        ],
        out_specs=pl.BlockSpec((sub, lanes), lambda i: (i, 0)),
        out_shape=jax.ShapeDtypeStruct((rows, lanes), jnp.float32),
        interpret=interpret,
    )(attr_t, wb)
    return out


def _gather_mul_sc(scores, x_flat, idx_flat, *, chunk=8192, interpret=False):
    total = x_flat.shape[0]
    n_cc = scores.shape[0]
    try:
        info = plsc.get_sparse_core_info()
        nc, ns = info.num_cores, info.num_subcores
    except ValueError:
        nc, ns = 2, 16
    nw = nc * ns
    per_w = total // nw
    assert total % nw == 0 and per_w % chunk == 0
    n_chunks = per_w // chunk
    mesh = plsc.VectorSubcoreMesh(
        core_axis_name="c", subcore_axis_name="s", num_cores=nc, num_subcores=ns)

    assert n_chunks % 2 == 0 and n_cc % ns == 0

    cs = (chunk * 13) // 16
    ch = chunk - cs
    assert cs % 16 == 0 and ch % 16 == 0

    def body(scores_hbm, x_hbm, idx_hbm, out_hbm, tab_sh, idxs0, idxs1,
             idxh0, idxh1, x0, x1, gs0, gs1, gh0, gh1,
             sem_i, sem_x, sem_gs, sem_gh, sem_o):
        c = lax.axis_index("c")
        s = lax.axis_index("s")
        wid = s * nc + c
        base = wid * per_w
        idxs_b = (idxs0, idxs1)
        idxh_b = (idxh0, idxh1)
        x_b = (x0, x1)
        gs_b = (gs0, gs1)
        gh_b = (gh0, gh1)

        def start_in(k, b):
            off = base + k * chunk
            pltpu.async_copy(idx_hbm.at[pl.ds(off, cs)], idxs_b[b],
                             sem_i.at[b])
            pltpu.async_copy(idx_hbm.at[pl.ds(off + cs, ch)], idxh_b[b],
                             sem_i.at[b])
            pltpu.async_copy(x_hbm.at[pl.ds(off, chunk)], x_b[b], sem_x.at[b])

        def wait_in(b):
            pltpu.make_async_copy(idx_hbm.at[pl.ds(0, cs)], idxs_b[b],
                                  sem_i.at[b]).wait()
            pltpu.make_async_copy(idx_hbm.at[pl.ds(0, ch)], idxh_b[b],
                                  sem_i.at[b]).wait()
            pltpu.make_async_copy(x_hbm.at[pl.ds(0, chunk)], x_b[b],
                                  sem_x.at[b]).wait()

        def wait_out(b):
            pltpu.make_async_copy(gs_b[b], out_hbm.at[pl.ds(0, cs)],
                                  sem_o.at[b]).wait()
            pltpu.make_async_copy(gh_b[b], out_hbm.at[pl.ds(0, ch)],
                                  sem_o.at[b]).wait()

        start_in(0, 0)
        start_in(1, 1)
        seg = n_cc // ns
        pltpu.sync_copy(scores_hbm.at[pl.ds(s * seg, seg)],
                        tab_sh.at[pl.ds(s * seg, seg)])
        plsc.subcore_barrier()

        def chunk_pair(g, carry):
            for b in range(2):
                k = 2 * g + b
                wait_in(b)

                @pl.when(k >= 2)
                def _():
                    wait_out(b)

                pltpu.async_copy(tab_sh.at[idxs_b[b]], gs_b[b], sem_gs.at[b])
                pltpu.async_copy(scores_hbm.at[idxh_b[b]], gh_b[b],
                                 sem_gh.at[b])
                pltpu.make_async_copy(tab_sh.at[idxs_b[b]], gs_b[b],
                                      sem_gs.at[b]).wait()
                pltpu.make_async_copy(scores_hbm.at[idxh_b[b]], gh_b[b],
                                      sem_gh.at[b]).wait()

                def mul_s(i, carry2):
                    sl = pl.ds(i * 16, 16)
                    gs_b[b][sl] = gs_b[b][sl] * x_b[b][sl]
                    return carry2

                def mul_h(i, carry2):
                    sl = pl.ds(i * 16, 16)
                    gh_b[b][sl] = gh_b[b][sl] * x_b[b][pl.ds(cs + i * 16, 16)]
                    return carry2

                lax.fori_loop(0, cs // 16, mul_s, 0, unroll=8)
                lax.fori_loop(0, ch // 16, mul_h, 0, unroll=8)
                off = base + k * chunk
                pltpu.async_copy(gs_b[b], out_hbm.at[pl.ds(off, cs)],
                                 sem_o.at[b])
                pltpu.async_copy(gh_b[b], out_hbm.at[pl.ds(off + cs, ch)],
                                 sem_o.at[b])

                @pl.when(k + 2 < n_chunks)
                def _():
                    start_in(k + 2, b)

            return carry

        lax.fori_loop(0, n_chunks // 2, chunk_pair, 0)
        wait_out(0)
        wait_out(1)

    fn = pl.kernel(
        body,
        out_type=jax.ShapeDtypeStruct((total,), jnp.float32),
        mesh=mesh,
        scratch_types=[
            pltpu.VMEM_SHARED((n_cc,), jnp.float32),
            pltpu.VMEM((cs,), jnp.int32),
            pltpu.VMEM((cs,), jnp.int32),
            pltpu.VMEM((ch,), jnp.int32),
            pltpu.VMEM((ch,), jnp.int32),
            pltpu.VMEM((chunk,), jnp.float32),
            pltpu.VMEM((chunk,), jnp.float32),
            pltpu.VMEM((cs,), jnp.float32),
            pltpu.VMEM((cs,), jnp.float32),
            pltpu.VMEM((ch,), jnp.float32),
            pltpu.VMEM((ch,), jnp.float32),
            pltpu.SemaphoreType.DMA((2,)),
            pltpu.SemaphoreType.DMA((2,)),
            pltpu.SemaphoreType.DMA((2,)),
            pltpu.SemaphoreType.DMA((2,)),
            pltpu.SemaphoreType.DMA((2,)),
        ],
        interpret=interpret,
    )
    return fn(scores, x_flat, idx_flat)


def kernel(input, attributes, pixel_cc, weight, bias):
    h, w = input.shape
    n = attributes.shape[0]
    n_pad = 1 << 20
    rows = lanes = 1024
    attr_p = jnp.concatenate(
        [attributes, jnp.ones((n_pad - n, 15), jnp.float32)], axis=0)
    attr_t = attr_p.T.reshape(15, rows, lanes)
    scores = _compute_scores(attr_t, weight, bias).reshape(-1)
    idx = pixel_cc.reshape(-1).astype(jnp.int32)
    x = input.reshape(-1)
    out = _gather_mul_sc(scores, x, idx)
    return out.reshape(h, w)


# speedup vs baseline: 1.4097x; 1.2053x over previous
import functools

import jax
import jax.numpy as jnp
from jax import lax
from jax.experimental import pallas as pl
from jax.experimental.pallas import tpu as pltpu
from jax.experimental.pallas import tpu_sc as plsc


def _scores_body(a_ref, wb_ref, o_ref):
    eps = 1e-10

    def r(k):
        return a_ref[k]

    def w(k):
        return wb_ref[k]

    z = w(0) * r(0) + w(1) * r(1) + w(2) * r(2) + w(3) * r(3)
    z = z + w(4) * jnp.log(r(4))
    for j in range(9):
        x = r(6 + j)
        z = z + w(5 + j) * (jnp.log(jnp.abs(x) + eps) * jnp.sign(x))
    z = z + w(14) * (jnp.sqrt(r(7)) / (jnp.sqrt(r(6)) + eps))
    z = z + w(15) * jnp.cos(r(5))
    z = z + w(16) * jnp.sin(r(5))
    z = z + w(17)
    o_ref[...] = 1.0 / (1.0 + jnp.exp(-z))


def _compute_scores(attr_t, weight, bias, *, interpret=False):
    _, rows, lanes = attr_t.shape
    sub = 32
    wb = jnp.concatenate([weight[:, 0], bias])
    out = pl.pallas_call(
        _scores_body,
        grid=(rows // sub,),
        in_specs=[
            pl.BlockSpec((15, sub, lanes), lambda i: (0, i, 0)),
            pl.BlockSpec(memory_space=pltpu.SMEM),
        ],
        out_specs=pl.BlockSpec((sub, lanes), lambda i: (i, 0)),
        out_shape=jax.ShapeDtypeStruct((rows, lanes), jnp.float32),
        interpret=interpret,
    )(attr_t, wb)
    return out


def _gather_mul_sc(scores, x_flat, idx_flat, *, chunk=8192, interpret=False):
    total = x_flat.shape[0]
    n_cc = scores.shape[0]
    try:
        info = plsc.get_sparse_core_info()
        nc, ns = info.num_cores, info.num_subcores
    except ValueError:
        nc, ns = 2, 16
    nw = nc * ns
    per_w = total // nw
    assert total % nw == 0 and per_w % chunk == 0
    n_chunks = per_w // chunk
    mesh = plsc.VectorSubcoreMesh(
        core_axis_name="c", subcore_axis_name="s", num_cores=nc, num_subcores=ns)

    assert n_chunks % 2 == 0 and n_cc % ns == 0

    def body(scores_hbm, x_hbm, idx_hbm, out_hbm, tab_sh, idx0, idx1, x0, x1,
             g0, g1, sem_i, sem_x, sem_g, sem_o):
        c = lax.axis_index("c")
        s = lax.axis_index("s")
        wid = s * nc + c
        base = wid * per_w
        idx_b = (idx0, idx1)
        x_b = (x0, x1)
        g_b = (g0, g1)

        def start_in(k, b):
            off = base + k * chunk
            pltpu.async_copy(idx_hbm.at[pl.ds(off, chunk)], idx_b[b],
                             sem_i.at[b])
            pltpu.async_copy(x_hbm.at[pl.ds(off, chunk)], x_b[b], sem_x.at[b])

        def wait_in(b):
            pltpu.make_async_copy(idx_hbm.at[pl.ds(0, chunk)], idx_b[b],
                                  sem_i.at[b]).wait()
            pltpu.make_async_copy(x_hbm.at[pl.ds(0, chunk)], x_b[b],
                                  sem_x.at[b]).wait()

        def wait_out(b):
            pltpu.make_async_copy(g_b[b], out_hbm.at[pl.ds(0, chunk)],
                                  sem_o.at[b]).wait()

        def start_gather(b):
            pltpu.async_copy(tab_sh.at[idx_b[b]], g_b[b], sem_g.at[b])

        def wait_gather(b):
            pltpu.make_async_copy(tab_sh.at[idx_b[b]], g_b[b],
                                  sem_g.at[b]).wait()

        start_in(0, 0)
        start_in(1, 1)
        seg = n_cc // ns
        pltpu.sync_copy(scores_hbm.at[pl.ds(s * seg, seg)],
                        tab_sh.at[pl.ds(s * seg, seg)])
        plsc.subcore_barrier()
        wait_in(0)
        start_gather(0)

        def chunk_pair(g, carry):
            for b in range(2):
                k = 2 * g + b
                b2 = 1 - b
                wait_gather(b)

                @pl.when(k + 1 < n_chunks)
                def _():
                    wait_in(b2)

                    @pl.when(k >= 1)
                    def _():
                        wait_out(b2)

                    start_gather(b2)

                def mul(i, carry2):
                    sl = pl.ds(i * 16, 16)
                    g_b[b][sl] = g_b[b][sl] * x_b[b][sl]
                    return carry2

                lax.fori_loop(0, chunk // 16, mul, 0, unroll=8)
                off = base + k * chunk
                pltpu.async_copy(g_b[b], out_hbm.at[pl.ds(off, chunk)],
                                 sem_o.at[b])

                @pl.when(k + 2 < n_chunks)
                def _():
                    start_in(k + 2, b)

            return carry

        lax.fori_loop(0, n_chunks // 2, chunk_pair, 0)
        wait_out(0)
        wait_out(1)

    fn = pl.kernel(
        body,
        out_type=jax.ShapeDtypeStruct((total,), jnp.float32),
        mesh=mesh,
        scratch_types=[
            pltpu.VMEM_SHARED((n_cc,), jnp.float32),
            pltpu.VMEM((chunk,), jnp.int32),
            pltpu.VMEM((chunk,), jnp.int32),
            pltpu.VMEM((chunk,), jnp.float32),
            pltpu.VMEM((chunk,), jnp.float32),
            pltpu.VMEM((chunk,), jnp.float32),
            pltpu.VMEM((chunk,), jnp.float32),
            pltpu.SemaphoreType.DMA((2,)),
            pltpu.SemaphoreType.DMA((2,)),
            pltpu.SemaphoreType.DMA((2,)),
            pltpu.SemaphoreType.DMA((2,)),
        ],
        interpret=interpret,
    )
    return fn(scores, x_flat, idx_flat)


def kernel(input, attributes, pixel_cc, weight, bias):
    h, w = input.shape
    n = attributes.shape[0]
    n_pad = 1 << 20
    rows = lanes = 1024
    attr_p = jnp.concatenate(
        [attributes, jnp.ones((n_pad - n, 15), jnp.float32)], axis=0)
    attr_t = attr_p.T.reshape(15, rows, lanes)
    scores = _compute_scores(attr_t, weight, bias).reshape(-1)
    idx = pixel_cc.reshape(-1).astype(jnp.int32)
    x = input.reshape(-1)
    out = _gather_mul_sc(scores, x, idx)
    return out.reshape(h, w)


# speedup vs baseline: 1.4171x; 1.0052x over previous
import functools

import jax
import jax.numpy as jnp
from jax import lax
from jax.experimental import pallas as pl
from jax.experimental.pallas import tpu as pltpu
from jax.experimental.pallas import tpu_sc as plsc


def _scores_body(a_ref, wb_ref, o_ref):
    eps = 1e-10

    def r(k):
        return a_ref[k]

    def w(k):
        return wb_ref[k]

    z = w(0) * r(0) + w(1) * r(1) + w(2) * r(2) + w(3) * r(3)
    z = z + w(4) * jnp.log(r(4))
    for j in range(9):
        x = r(6 + j)
        z = z + w(5 + j) * (jnp.log(jnp.abs(x) + eps) * jnp.sign(x))
    z = z + w(14) * (jnp.sqrt(r(7)) / (jnp.sqrt(r(6)) + eps))
    z = z + w(15) * jnp.cos(r(5))
    z = z + w(16) * jnp.sin(r(5))
    z = z + w(17)
    o_ref[...] = 1.0 / (1.0 + jnp.exp(-z))


def _compute_scores(attr_t, weight, bias, *, interpret=False):
    _, rows, lanes = attr_t.shape
    sub = 64
    wb = jnp.concatenate([weight[:, 0], bias])
    out = pl.pallas_call(
        _scores_body,
        grid=(rows // sub,),
        in_specs=[
            pl.BlockSpec((15, sub, lanes), lambda i: (0, i, 0)),
            pl.BlockSpec(memory_space=pltpu.SMEM),
        ],
        out_specs=pl.BlockSpec((sub, lanes), lambda i: (i, 0)),
        out_shape=jax.ShapeDtypeStruct((rows, lanes), jnp.float32),
        interpret=interpret,
    )(attr_t, wb)
    return out


def _gather_mul_sc(scores, x_flat, idx_flat, *, chunk=8192, interpret=False):
    total = x_flat.shape[0]
    n_cc = scores.shape[0]
    try:
        info = plsc.get_sparse_core_info()
        nc, ns = info.num_cores, info.num_subcores
    except ValueError:
        nc, ns = 2, 16
    nw = nc * ns
    per_w = total // nw
    assert total % nw == 0 and per_w % chunk == 0
    n_chunks = per_w // chunk
    mesh = plsc.VectorSubcoreMesh(
        core_axis_name="c", subcore_axis_name="s", num_cores=nc, num_subcores=ns)

    assert n_chunks % 2 == 0 and n_cc % ns == 0

    def body(scores_hbm, x_hbm, idx_hbm, out_hbm, tab_sh, idx0, idx1, x0, x1,
             g0, g1, sem_i, sem_x, sem_g, sem_o):
        c = lax.axis_index("c")
        s = lax.axis_index("s")
        wid = s * nc + c
        base = wid * per_w
        idx_b = (idx0, idx1)
        x_b = (x0, x1)
        g_b = (g0, g1)

        def start_in(k, b):
            off = base + k * chunk
            pltpu.async_copy(idx_hbm.at[pl.ds(off, chunk)], idx_b[b],
                             sem_i.at[b])
            pltpu.async_copy(x_hbm.at[pl.ds(off, chunk)], x_b[b], sem_x.at[b])

        def wait_in(b):
            pltpu.make_async_copy(idx_hbm.at[pl.ds(0, chunk)], idx_b[b],
                                  sem_i.at[b]).wait()
            pltpu.make_async_copy(x_hbm.at[pl.ds(0, chunk)], x_b[b],
                                  sem_x.at[b]).wait()

        def wait_out(b):
            pltpu.make_async_copy(g_b[b], out_hbm.at[pl.ds(0, chunk)],
                                  sem_o.at[b]).wait()

        def start_gather(b):
            pltpu.async_copy(tab_sh.at[idx_b[b]], g_b[b], sem_g.at[b])

        def wait_gather(b):
            pltpu.make_async_copy(tab_sh.at[idx_b[b]], g_b[b],
                                  sem_g.at[b]).wait()

        start_in(0, 0)
        start_in(1, 1)
        seg = n_cc // ns
        pltpu.sync_copy(scores_hbm.at[pl.ds(s * seg, seg)],
                        tab_sh.at[pl.ds(s * seg, seg)])
        plsc.subcore_barrier()
        wait_in(0)
        start_gather(0)

        def chunk_pair(g, carry):
            for b in range(2):
                k = 2 * g + b
                b2 = 1 - b
                wait_gather(b)

                @pl.when(k + 1 < n_chunks)
                def _():
                    wait_in(b2)

                    @pl.when(k >= 1)
                    def _():
                        wait_out(b2)

                    start_gather(b2)

                def mul(i, carry2):
                    sl = pl.ds(i * 16, 16)
                    g_b[b][sl] = g_b[b][sl] * x_b[b][sl]
                    return carry2

                lax.fori_loop(0, chunk // 16, mul, 0, unroll=16)
                off = base + k * chunk
                pltpu.async_copy(g_b[b], out_hbm.at[pl.ds(off, chunk)],
                                 sem_o.at[b])

                @pl.when(k + 2 < n_chunks)
                def _():
                    start_in(k + 2, b)

            return carry

        lax.fori_loop(0, n_chunks // 2, chunk_pair, 0)
        wait_out(0)
        wait_out(1)

    fn = pl.kernel(
        body,
        out_type=jax.ShapeDtypeStruct((total,), jnp.float32),
        mesh=mesh,
        scratch_types=[
            pltpu.VMEM_SHARED((n_cc,), jnp.float32),
            pltpu.VMEM((chunk,), jnp.int32),
            pltpu.VMEM((chunk,), jnp.int32),
            pltpu.VMEM((chunk,), jnp.float32),
            pltpu.VMEM((chunk,), jnp.float32),
            pltpu.VMEM((chunk,), jnp.float32),
            pltpu.VMEM((chunk,), jnp.float32),
            pltpu.SemaphoreType.DMA((2,)),
            pltpu.SemaphoreType.DMA((2,)),
            pltpu.SemaphoreType.DMA((2,)),
            pltpu.SemaphoreType.DMA((2,)),
        ],
        interpret=interpret,
    )
    return fn(scores, x_flat, idx_flat)


def kernel(input, attributes, pixel_cc, weight, bias):
    h, w = input.shape
    n = attributes.shape[0]
    n_pad = 1 << 20
    rows = lanes = 1024
    attr_p = jnp.concatenate(
        [attributes, jnp.ones((n_pad - n, 15), jnp.float32)], axis=0)
    attr_t = attr_p.T.reshape(15, rows, lanes)
    scores = _compute_scores(attr_t, weight, bias).reshape(-1)
    idx = pixel_cc.reshape(-1).astype(jnp.int32)
    x = input.reshape(-1)
    out = _gather_mul_sc(scores, x, idx)
    return out.reshape(h, w)


# speedup vs baseline: 1.4174x; 1.0002x over previous
import jax
import jax.numpy as jnp
from jax import lax
from jax.experimental import pallas as pl
from jax.experimental.pallas import tpu as pltpu
from jax.experimental.pallas import tpu_sc as plsc


def _scores_body(a_ref, wb_ref, o_ref):
    eps = 1e-10

    def r(k):
        return a_ref[k]

    def w(k):
        return wb_ref[k]

    z = w(0) * r(0) + w(1) * r(1) + w(2) * r(2) + w(3) * r(3)
    z = z + w(4) * jnp.log(r(4))
    for j in range(9):
        x = r(6 + j)
        z = z + w(5 + j) * (jnp.log(jnp.abs(x) + eps) * jnp.sign(x))
    z = z + w(14) * (jnp.sqrt(r(7)) / (jnp.sqrt(r(6)) + eps))
    z = z + w(15) * jnp.cos(r(5))
    z = z + w(16) * jnp.sin(r(5))
    z = z + w(17)
    o_ref[...] = 1.0 / (1.0 + jnp.exp(-z))


def _compute_scores(attr_t, weight, bias, *, interpret=False):
    _, rows, lanes = attr_t.shape
    sub = 64
    wb = jnp.concatenate([weight[:, 0], bias])
    out = pl.pallas_call(
        _scores_body,
        grid=(rows // sub,),
        in_specs=[
            pl.BlockSpec((15, sub, lanes), lambda i: (0, i, 0)),
            pl.BlockSpec(memory_space=pltpu.SMEM),
        ],
        out_specs=pl.BlockSpec((sub, lanes), lambda i: (i, 0)),
        out_shape=jax.ShapeDtypeStruct((rows, lanes), jnp.float32),
        interpret=interpret,
    )(attr_t, wb)
    return out


def _gather_mul_sc(scores, x_flat, idx_flat, *, chunk=8192, interpret=False):
    total = x_flat.shape[0]
    n_cc = scores.shape[0]
    try:
        info = plsc.get_sparse_core_info()
        nc, ns = info.num_cores, info.num_subcores
    except ValueError:
        nc, ns = 2, 16
    nw = nc * ns
    per_w = total // nw
    assert total % nw == 0 and per_w % chunk == 0
    n_chunks = per_w // chunk
    mesh = plsc.VectorSubcoreMesh(
        core_axis_name="c", subcore_axis_name="s", num_cores=nc, num_subcores=ns)

    assert n_chunks % 2 == 0 and n_cc % ns == 0

    def body(scores_hbm, x_hbm, idx_hbm, out_hbm, tab_sh, idx0, idx1, x0, x1,
             g0, g1, sem_i, sem_x, sem_g, sem_o):
        c = lax.axis_index("c")
        s = lax.axis_index("s")
        wid = s * nc + c
        base = wid * per_w
        idx_b = (idx0, idx1)
        x_b = (x0, x1)
        g_b = (g0, g1)

        def start_in(k, b):
            off = base + k * chunk
            pltpu.async_copy(idx_hbm.at[pl.ds(off, chunk)], idx_b[b],
                             sem_i.at[b])
            pltpu.async_copy(x_hbm.at[pl.ds(off, chunk)], x_b[b], sem_x.at[b])

        def wait_in(b):
            pltpu.make_async_copy(idx_hbm.at[pl.ds(0, chunk)], idx_b[b],
                                  sem_i.at[b]).wait()
            pltpu.make_async_copy(x_hbm.at[pl.ds(0, chunk)], x_b[b],
                                  sem_x.at[b]).wait()

        def wait_out(b):
            pltpu.make_async_copy(g_b[b], out_hbm.at[pl.ds(0, chunk)],
                                  sem_o.at[b]).wait()

        def start_gather(b):
            pltpu.async_copy(tab_sh.at[idx_b[b]], g_b[b], sem_g.at[b])

        def wait_gather(b):
            pltpu.make_async_copy(tab_sh.at[idx_b[b]], g_b[b],
                                  sem_g.at[b]).wait()

        start_in(0, 0)
        start_in(1, 1)
        seg = n_cc // ns
        pltpu.sync_copy(scores_hbm.at[pl.ds(s * seg, seg)],
                        tab_sh.at[pl.ds(s * seg, seg)])
        plsc.subcore_barrier()
        wait_in(0)
        start_gather(0)

        def chunk_pair(g, carry):
            for b in range(2):
                k = 2 * g + b
                b2 = 1 - b
                wait_gather(b)

                @pl.when(k + 1 < n_chunks)
                def _():
                    wait_in(b2)

                    @pl.when(k >= 1)
                    def _():
                        wait_out(b2)

                    start_gather(b2)

                def mul(i, carry2):
                    sl = pl.ds(i * 16, 16)
                    g_b[b][sl] = g_b[b][sl] * x_b[b][sl]
                    return carry2

                lax.fori_loop(0, chunk // 16, mul, 0, unroll=16)
                off = base + k * chunk
                pltpu.async_copy(g_b[b], out_hbm.at[pl.ds(off, chunk)],
                                 sem_o.at[b])

                @pl.when(k + 2 < n_chunks)
                def _():
                    start_in(k + 2, b)

            return carry

        lax.fori_loop(0, n_chunks // 2, chunk_pair, 0)
        wait_out(0)
        wait_out(1)

    fn = pl.kernel(
        body,
        out_type=jax.ShapeDtypeStruct((total,), jnp.float32),
        mesh=mesh,
        scratch_types=[
            pltpu.VMEM_SHARED((n_cc,), jnp.float32),
            pltpu.VMEM((chunk,), jnp.int32),
            pltpu.VMEM((chunk,), jnp.int32),
            pltpu.VMEM((chunk,), jnp.float32),
            pltpu.VMEM((chunk,), jnp.float32),
            pltpu.VMEM((chunk,), jnp.float32),
            pltpu.VMEM((chunk,), jnp.float32),
            pltpu.SemaphoreType.DMA((2,)),
            pltpu.SemaphoreType.DMA((2,)),
            pltpu.SemaphoreType.DMA((2,)),
            pltpu.SemaphoreType.DMA((2,)),
        ],
        interpret=interpret,
    )
    return fn(scores, x_flat, idx_flat)


def kernel(input, attributes, pixel_cc, weight, bias):
    h, w = input.shape
    n = attributes.shape[0]
    n_pad = 1 << 20
    rows = lanes = 1024
    attr_p = jnp.concatenate(
        [attributes, jnp.ones((n_pad - n, 15), jnp.float32)], axis=0)
    attr_t = attr_p.T.reshape(15, rows, lanes)
    scores = _compute_scores(attr_t, weight, bias).reshape(-1)
    idx = pixel_cc.reshape(-1).astype(jnp.int32)
    x = input.reshape(-1)
    out = _gather_mul_sc(scores, x, idx)
    return out.reshape(h, w)
